# baseline (speedup 1.0000x reference)
import functools

import jax
import jax.numpy as jnp
from jax import lax
from jax.experimental import pallas as pl
from jax.experimental.pallas import tpu as pltpu
from jax.experimental.pallas import tpu_sc as plsc

DIM = 64
BATCH = 4096
SEQ = 200
VOCAB = 1000000

NC = 2
NS = 16
NW = NC * NS
BTILE = BATCH // NW
NBUF = 4

_mesh = plsc.VectorSubcoreMesh(core_axis_name="c", subcore_axis_name="s")


@functools.partial(
    pl.kernel,
    mesh=_mesh,
    out_type=jax.ShapeDtypeStruct((SEQ, DIM, BATCH), jnp.float32),
    scratch_types=[
        pltpu.VMEM((SEQ, BTILE), jnp.int32),
        *[pltpu.VMEM((BTILE, 2 * DIM), jnp.float32) for _ in range(NBUF)],
        *[pltpu.VMEM((BTILE,), jnp.int32) for _ in range(NBUF)],
        *[pltpu.VMEM((BTILE,), jnp.int32) for _ in range(NBUF)],
        *[pltpu.VMEM((DIM, BTILE), jnp.float32) for _ in range(2)],
        *[pltpu.SemaphoreType.DMA for _ in range(NBUF)],
        *[pltpu.SemaphoreType.DMA for _ in range(2)],
    ],
    compiler_params=pltpu.CompilerParams(
        use_tc_tiling_on_sc=True, needs_layout_passes=False),
)
def _emb(idxT_hbm, tpair_hbm, out_hbm, idxblk,
         b0, b1, b2, b3, p0, p1, p2, p3, h0, h1, h2, h3, t0, t1,
         g0, g1, g2, g3, o0, o1):
    bufs = [b0, b1, b2, b3]
    pids = [p0, p1, p2, p3]
    hofs = [h0, h1, h2, h3]
    touts = [t0, t1]
    gsems = [g0, g1, g2, g3]
    osems = [o0, o1]

    wid = lax.axis_index("s") * NC + lax.axis_index("c")
    base_b = wid * BTILE

    pltpu.sync_copy(idxT_hbm.at[:, pl.ds(base_b, BTILE)], idxblk)

    def prep(u, pid, hof):
        @pl.loop(0, BTILE // 16)
        def _(g):
            v = idxblk[u, pl.ds(g * 16, 16)]
            pid[pl.ds(g * 16, 16)] = lax.shift_right_logical(v, 1)
            hof[pl.ds(g * 16, 16)] = lax.shift_left(lax.bitwise_and(v, 1), 6)

    def start_gather(buf, pid, sem):
        pltpu.async_copy(tpair_hbm.at[pid], buf, sem)

    def wait_gather(buf, pid, sem):
        pltpu.make_async_copy(tpair_hbm.at[pid], buf, sem).wait()

    def extract(buf, hof, tout):
        @pl.loop(0, BTILE // 16)
        def _(g):
            h16 = hof[pl.ds(g * 16, 16)]
            t16 = lax.iota(jnp.int32, 16) + g * 16
            for d in range(DIM):
                tout[d, pl.ds(g * 16, 16)] = plsc.load_gather(
                    buf, [t16, h16 + d])

    def out_slab(u):
        return out_hbm.at[u, :, pl.ds(base_b, BTILE)]

    def start_scatter(u, tout, sem):
        pltpu.async_copy(tout, out_slab(u), sem)

    def wait_scatter(u, tout, sem):
        pltpu.make_async_copy(tout, out_slab(u), sem).wait()

    for j in range(NBUF):
        prep(j, pids[j], hofs[j])
        start_gather(bufs[j], pids[j], gsems[j])

    @pl.loop(0, SEQ, step=NBUF)
    def _ring(s):
        for j in range(NBUF):
            u = s + j
            wait_gather(bufs[j], pids[j], gsems[j])

            @pl.when(u >= 2)
            def _():
                wait_scatter(u - 2, touts[j % 2], osems[j % 2])

            extract(bufs[j], hofs[j], touts[j % 2])
            start_scatter(u, touts[j % 2], osems[j % 2])

            @pl.when(u + NBUF < SEQ)
            def _():
                prep(u + NBUF, pids[j], hofs[j])
                start_gather(bufs[j], pids[j], gsems[j])

    wait_scatter(SEQ - 2, touts[0], osems[0])
    wait_scatter(SEQ - 1, touts[1], osems[1])


def kernel(idx, table):
    idxT = jnp.transpose(idx)
    tpair = jnp.reshape(table, (VOCAB // 2, 2 * DIM))
    kout = _emb(idxT, tpair)
    return jnp.transpose(kout, (2, 0, 1))


# speedup vs baseline: 1.7104x; 1.7104x over previous
import functools

import jax
import jax.numpy as jnp
from jax import lax
from jax.experimental import pallas as pl
from jax.experimental.pallas import tpu as pltpu
from jax.experimental.pallas import tpu_sc as plsc

DIM = 64
BATCH = 4096
SEQ = 200
VOCAB = 1000000

NC = 2
NS = 16
NW = NC * NS
BTILE = BATCH // NW
NBUF = 4

_mesh = plsc.VectorSubcoreMesh(core_axis_name="c", subcore_axis_name="s")


@functools.partial(
    pl.kernel,
    mesh=_mesh,
    out_type=jax.ShapeDtypeStruct((SEQ, DIM, BATCH), jnp.float32),
    scratch_types=[
        pltpu.VMEM((SEQ, BTILE), jnp.int32),
        *[pltpu.VMEM((BTILE, 2 * DIM), jnp.float32) for _ in range(NBUF)],
        *[pltpu.VMEM((BTILE,), jnp.int32) for _ in range(NBUF)],
        *[pltpu.VMEM((BTILE,), jnp.int32) for _ in range(NBUF)],
        *[pltpu.VMEM((DIM, BTILE), jnp.float32) for _ in range(2)],
        *[pltpu.SemaphoreType.DMA for _ in range(NBUF)],
        *[pltpu.SemaphoreType.DMA for _ in range(2)],
    ],
    compiler_params=pltpu.CompilerParams(
        use_tc_tiling_on_sc=True, needs_layout_passes=False),
)
def _emb(idxT_hbm, tpair_hbm, out_hbm, idxblk,
         b0, b1, b2, b3, p0, p1, p2, p3, h0, h1, h2, h3, t0, t1,
         g0, g1, g2, g3, o0, o1):
    bufs = [b0, b1, b2, b3]
    pids = [p0, p1, p2, p3]
    hofs = [h0, h1, h2, h3]
    touts = [t0, t1]
    gsems = [g0, g1, g2, g3]
    osems = [o0, o1]

    wid = lax.axis_index("s") * NC + lax.axis_index("c")
    base_b = wid * BTILE

    pltpu.sync_copy(idxT_hbm.at[:, pl.ds(base_b, BTILE)], idxblk)

    def prep(u, pid, hof):
        @pl.loop(0, BTILE // 16)
        def _(g):
            v = idxblk[u, pl.ds(g * 16, 16)]
            pid[pl.ds(g * 16, 16)] = lax.shift_right_logical(v, 1)
            hof[pl.ds(g * 16, 16)] = lax.shift_left(lax.bitwise_and(v, 1), 6)

    def start_gather(buf, pid, sem):
        pltpu.async_copy(tpair_hbm.at[pid], buf, sem)

    def wait_gather(buf, pid, sem):
        pltpu.make_async_copy(tpair_hbm.at[pid], buf, sem).wait()

    def extract(buf, hof, tout):
        @pl.loop(0, BTILE // 16)
        def _(g):
            i16 = lax.iota(jnp.int32, 16)
            t16 = i16 + g * 16
            h16 = hof[pl.ds(g * 16, 16)]
            for k in range(16):
                rot = lax.bitwise_and(i16 + k, 15)
                hrot = h16 + rot
                for dblk in range(DIM // 16):
                    val = plsc.load_gather(buf, [t16, hrot + dblk * 16])
                    plsc.store_scatter(tout, [rot + dblk * 16, t16], val)

    def out_slab(u):
        return out_hbm.at[u, :, pl.ds(base_b, BTILE)]

    def start_scatter(u, tout, sem):
        pltpu.async_copy(tout, out_slab(u), sem)

    def wait_scatter(u, tout, sem):
        pltpu.make_async_copy(tout, out_slab(u), sem).wait()

    for j in range(NBUF):
        prep(j, pids[j], hofs[j])
        start_gather(bufs[j], pids[j], gsems[j])

    @pl.loop(0, SEQ, step=NBUF)
    def _ring(s):
        for j in range(NBUF):
            u = s + j
            wait_gather(bufs[j], pids[j], gsems[j])

            @pl.when(u >= 2)
            def _():
                wait_scatter(u - 2, touts[j % 2], osems[j % 2])

            extract(bufs[j], hofs[j], touts[j % 2])
            start_scatter(u, touts[j % 2], osems[j % 2])

            @pl.when(u + NBUF < SEQ)
            def _():
                prep(u + NBUF, pids[j], hofs[j])
                start_gather(bufs[j], pids[j], gsems[j])

    wait_scatter(SEQ - 2, touts[0], osems[0])
    wait_scatter(SEQ - 1, touts[1], osems[1])


def kernel(idx, table):
    idxT = jnp.transpose(idx)
    tpair = jnp.reshape(table, (VOCAB // 2, 2 * DIM))
    kout = _emb(idxT, tpair)
    return jnp.transpose(kout, (2, 0, 1))
